# baseline (speedup 1.0000x reference)
import jax
import jax.numpy as jnp
from jax import lax
from jax.experimental import pallas as pl
from jax.experimental.pallas import tpu as pltpu
from jax.experimental.pallas import tpu_sc as plsc

N_NODES = 10000
DIM = 128
N_EDGES = 320000

NC = 2
NS = 16
NW = NC * NS

EB = 125
BLK_PER_TILE = 80
NBLK = NW * BLK_PER_TILE
AGG_ROWS = N_NODES
ROWS_PER_TILE = 624
ROWS_TAIL = N_NODES - NS * ROWS_PER_TILE
NBUF = 2
NPHASE = 5
PHB = BLK_PER_TILE // NPHASE


def _segsum_body(x_hbm, ei_hbm, zeros_hbm, out_hbm,
                 idx_s, idx_d, rows, agg_sh, sems, isem_s, isem_d):
    c = lax.axis_index("c")
    s = lax.axis_index("s")
    wid = c * NS + s
    iboff = pl.multiple_of(wid * BLK_PER_TILE, 8)

    def stage(p):
        q = p % 2
        pltpu.async_copy(ei_hbm.at[0, pl.ds(iboff + p * PHB, PHB)],
                         idx_s.at[q], isem_s[q])
        pltpu.async_copy(ei_hbm.at[1, pl.ds(iboff + p * PHB, PHB)],
                         idx_d.at[q], isem_d[q])

    def stage_wait(p):
        q = p % 2
        pltpu.make_async_copy(ei_hbm.at[0, pl.ds(iboff + p * PHB, PHB)],
                              idx_s.at[q], isem_s[q]).wait()
        pltpu.make_async_copy(ei_hbm.at[1, pl.ds(iboff + p * PHB, PHB)],
                              idx_d.at[q], isem_d[q]).wait()

    stage(0)

    base = pl.multiple_of(s * ROWS_PER_TILE, 8)

    @pl.when(c == 0)
    def _():
        pltpu.sync_copy(x_hbm.at[pl.ds(base, ROWS_PER_TILE)],
                        agg_sh.at[pl.ds(base, ROWS_PER_TILE)])

        @pl.when(s == NS - 1)
        def _():
            pltpu.sync_copy(x_hbm.at[pl.ds(NS * ROWS_PER_TILE, ROWS_TAIL)],
                            agg_sh.at[pl.ds(NS * ROWS_PER_TILE, ROWS_TAIL)])

    @pl.when(c == 1)
    def _():
        pltpu.sync_copy(zeros_hbm.at[pl.ds(base, ROWS_PER_TILE)],
                        agg_sh.at[pl.ds(base, ROWS_PER_TILE)])

        @pl.when(s == NS - 1)
        def _():
            pltpu.sync_copy(zeros_hbm.at[pl.ds(NS * ROWS_PER_TILE,
                                               ROWS_TAIL)],
                            agg_sh.at[pl.ds(NS * ROWS_PER_TILE, ROWS_TAIL)])

    plsc.subcore_barrier()

    for p in range(NPHASE):
        q = p % 2
        stage_wait(p)
        if p + 1 < NPHASE:
            stage(p + 1)

        def gather(b, blk):
            pltpu.async_copy(x_hbm.at[idx_s.at[q, blk]], rows.at[b], sems[b])

        for b in range(NBUF):
            gather(b, b)

        def body(j, carry):
            for b in range(NBUF):
                blk = j + b
                pltpu.make_async_copy(x_hbm.at[idx_s.at[q, blk]], rows.at[b],
                                      sems[b]).wait()
                pltpu.sync_copy(rows.at[b], agg_sh.at[idx_d.at[q, blk]],
                                add=True)

                @pl.when(blk + NBUF < PHB)
                def _():
                    gather(b, blk + NBUF)

            return carry

        lax.fori_loop(0, PHB // NBUF, lambda j, cr: body(j * NBUF, cr), 0)

    plsc.subcore_barrier()
    pltpu.sync_copy(agg_sh.at[pl.ds(base, ROWS_PER_TILE)],
                    out_hbm.at[c, pl.ds(base, ROWS_PER_TILE)])

    @pl.when(s == NS - 1)
    def _():
        pltpu.sync_copy(agg_sh.at[pl.ds(NS * ROWS_PER_TILE, ROWS_TAIL)],
                        out_hbm.at[c, pl.ds(NS * ROWS_PER_TILE, ROWS_TAIL)])


@jax.jit
def _sc_segsum(x, ei, zeros):
    mesh = plsc.VectorSubcoreMesh(core_axis_name="c", subcore_axis_name="s")
    return pl.kernel(
        _segsum_body,
        out_type=jax.ShapeDtypeStruct((NC, N_NODES, DIM), jnp.float32),
        mesh=mesh,
        scratch_types=[
            pltpu.VMEM((2, PHB, EB), jnp.int32),
            pltpu.VMEM((2, PHB, EB), jnp.int32),
            pltpu.VMEM((NBUF, EB, DIM), jnp.float32),
            pltpu.VMEM_SHARED((AGG_ROWS, DIM), jnp.float32),
            [pltpu.SemaphoreType.DMA] * NBUF,
            [pltpu.SemaphoreType.DMA] * 2,
            [pltpu.SemaphoreType.DMA] * 2,
        ],
    )(x, ei, zeros)


ROW_BLK = 1000


def _mlp_body(agg_ref, wa_ref, wb_ref, wc_ref, l_ref,
              hout_ref, head_ref, acc_ref):
    i = pl.program_id(0)
    z = agg_ref[0] + agg_ref[1]
    z = jnp.maximum(jnp.dot(z, wa_ref[...], preferred_element_type=jnp.float32), 0.0)
    z = jnp.maximum(jnp.dot(z, wb_ref[...], preferred_element_type=jnp.float32), 0.0)
    z = jnp.maximum(jnp.dot(z, wc_ref[...], preferred_element_type=jnp.float32), 0.0)
    hout_ref[...] = z

    @pl.when(i == 0)
    def _():
        acc_ref[...] = jnp.zeros_like(acc_ref)

    acc_ref[...] += jnp.sum(z, axis=0, keepdims=True)

    @pl.when(i == pl.num_programs(0) - 1)
    def _():
        head_ref[...] = jnp.dot(acc_ref[...] / N_NODES, l_ref[...],
                                preferred_element_type=jnp.float32)


def _final_body(agg_ref, wa_ref, wb_ref, wc_ref, l_ref,
                h0_ref, h1_ref, h2_ref, h3_ref, out_ref, acc_ref):
    i = pl.program_id(0)
    z = agg_ref[0] + agg_ref[1]
    z = jnp.maximum(jnp.dot(z, wa_ref[...], preferred_element_type=jnp.float32), 0.0)
    z = jnp.maximum(jnp.dot(z, wb_ref[...], preferred_element_type=jnp.float32), 0.0)
    z = jnp.maximum(jnp.dot(z, wc_ref[...], preferred_element_type=jnp.float32), 0.0)

    @pl.when(i == 0)
    def _():
        acc_ref[...] = jnp.zeros_like(acc_ref)

    acc_ref[...] += jnp.sum(z, axis=0, keepdims=True)

    @pl.when(i == pl.num_programs(0) - 1)
    def _():
        head = jnp.dot(acc_ref[...] / N_NODES, l_ref[...],
                       preferred_element_type=jnp.float32)
        total = head + h0_ref[...] + h1_ref[...] + h2_ref[...] + h3_ref[...]
        out_ref[...] = jax.nn.sigmoid(total)


def _tc_mlp(agg, wa, wb, wc, l):
    grid = N_NODES // ROW_BLK
    return pl.pallas_call(
        _mlp_body,
        grid=(grid,),
        in_specs=[
            pl.BlockSpec((NC, ROW_BLK, DIM), lambda i: (0, i, 0)),
            pl.BlockSpec((DIM, DIM), lambda i: (0, 0)),
            pl.BlockSpec((DIM, DIM), lambda i: (0, 0)),
            pl.BlockSpec((DIM, DIM), lambda i: (0, 0)),
            pl.BlockSpec((DIM, 1), lambda i: (0, 0)),
        ],
        out_specs=[
            pl.BlockSpec((ROW_BLK, DIM), lambda i: (i, 0)),
            pl.BlockSpec((1, 1), lambda i: (0, 0)),
        ],
        out_shape=[
            jax.ShapeDtypeStruct((N_NODES, DIM), jnp.float32),
            jax.ShapeDtypeStruct((1, 1), jnp.float32),
        ],
        scratch_shapes=[pltpu.VMEM((1, DIM), jnp.float32)],
    )(agg, wa, wb, wc, l)


def _tc_final(agg, wa, wb, wc, l, h0, h1, h2, h3):
    grid = N_NODES // ROW_BLK
    hspec = pl.BlockSpec((1, 1), lambda i: (0, 0))
    return pl.pallas_call(
        _final_body,
        grid=(grid,),
        in_specs=[
            pl.BlockSpec((NC, ROW_BLK, DIM), lambda i: (0, i, 0)),
            pl.BlockSpec((DIM, DIM), lambda i: (0, 0)),
            pl.BlockSpec((DIM, DIM), lambda i: (0, 0)),
            pl.BlockSpec((DIM, DIM), lambda i: (0, 0)),
            pl.BlockSpec((DIM, 1), lambda i: (0, 0)),
            hspec, hspec, hspec, hspec,
        ],
        out_specs=pl.BlockSpec((1, 1), lambda i: (0, 0)),
        out_shape=jax.ShapeDtypeStruct((1, 1), jnp.float32),
        scratch_shapes=[pltpu.VMEM((1, DIM), jnp.float32)],
    )(agg, wa, wb, wc, l, h0, h1, h2, h3)


def kernel(x, edge_index, Wa, Wb, Wc, L):
    ei = edge_index.reshape(2, NBLK, EB)
    zeros = jnp.zeros((AGG_ROWS, DIM), jnp.float32)

    h = x
    heads = []
    for i in range(4):
        agg = _sc_segsum(h, ei, zeros)
        h, head = _tc_mlp(agg, Wa[i], Wb[i], Wc[i], L[i])
        heads.append(head)

    agg = _sc_segsum(h, ei, zeros)
    out = _tc_final(agg, Wa[4], Wb[4], Wc[4], L[4], *heads)
    return out.reshape((1,))


# speedup vs baseline: 1.0270x; 1.0270x over previous
import jax
import jax.numpy as jnp
from jax import lax
from jax.experimental import pallas as pl
from jax.experimental.pallas import tpu as pltpu
from jax.experimental.pallas import tpu_sc as plsc

N_NODES = 10000
DIM = 128
N_EDGES = 320000

NC = 2
NS = 16
NW = NC * NS

EB = 125
BLK_PER_TILE = 80
NBLK = NW * BLK_PER_TILE
AGG_ROWS = N_NODES
ROWS_PER_TILE = 624
ROWS_TAIL = N_NODES - NS * ROWS_PER_TILE
NBUF = 2
NPHASE = 5
PHB = BLK_PER_TILE // NPHASE


def _segsum_body(x_hbm, ei_hbm, zeros_hbm, out_hbm,
                 idx_s, idx_d, rows, agg_sh, sems, isem_s, isem_d):
    c = lax.axis_index("c")
    s = lax.axis_index("s")
    wid = c * NS + s
    iboff = pl.multiple_of(wid * BLK_PER_TILE, 8)

    def stage(p):
        q = p % 2
        pltpu.async_copy(ei_hbm.at[0, pl.ds(iboff + p * PHB, PHB)],
                         idx_s.at[q], isem_s[q])
        pltpu.async_copy(ei_hbm.at[1, pl.ds(iboff + p * PHB, PHB)],
                         idx_d.at[q], isem_d[q])

    def stage_wait(p):
        q = p % 2
        pltpu.make_async_copy(ei_hbm.at[0, pl.ds(iboff + p * PHB, PHB)],
                              idx_s.at[q], isem_s[q]).wait()
        pltpu.make_async_copy(ei_hbm.at[1, pl.ds(iboff + p * PHB, PHB)],
                              idx_d.at[q], isem_d[q]).wait()

    stage(0)

    base = pl.multiple_of(s * ROWS_PER_TILE, 8)

    @pl.when(c == 0)
    def _():
        pltpu.sync_copy(x_hbm.at[pl.ds(base, ROWS_PER_TILE)],
                        agg_sh.at[pl.ds(base, ROWS_PER_TILE)])

        @pl.when(s == NS - 1)
        def _():
            pltpu.sync_copy(x_hbm.at[pl.ds(NS * ROWS_PER_TILE, ROWS_TAIL)],
                            agg_sh.at[pl.ds(NS * ROWS_PER_TILE, ROWS_TAIL)])

    @pl.when(c == 1)
    def _():
        pltpu.sync_copy(zeros_hbm.at[pl.ds(base, ROWS_PER_TILE)],
                        agg_sh.at[pl.ds(base, ROWS_PER_TILE)])

        @pl.when(s == NS - 1)
        def _():
            pltpu.sync_copy(zeros_hbm.at[pl.ds(NS * ROWS_PER_TILE,
                                               ROWS_TAIL)],
                            agg_sh.at[pl.ds(NS * ROWS_PER_TILE, ROWS_TAIL)])

    plsc.subcore_barrier()

    for p in range(NPHASE):
        q = p % 2
        stage_wait(p)
        if p + 1 < NPHASE:
            stage(p + 1)

        def gather(b, blk):
            pltpu.async_copy(x_hbm.at[idx_s.at[q, blk]], rows.at[b], sems[b])

        for b in range(NBUF):
            gather(b, b)

        def body(j, carry):
            for b in range(NBUF):
                blk = j + b
                pltpu.make_async_copy(x_hbm.at[idx_s.at[q, blk]], rows.at[b],
                                      sems[b]).wait()
                pltpu.sync_copy(rows.at[b], agg_sh.at[idx_d.at[q, blk]],
                                add=True)

                @pl.when(blk + NBUF < PHB)
                def _():
                    gather(b, blk + NBUF)

            return carry

        lax.fori_loop(0, PHB // NBUF, lambda j, cr: body(j * NBUF, cr), 0)

    plsc.subcore_barrier()
    pltpu.sync_copy(agg_sh.at[pl.ds(base, ROWS_PER_TILE)],
                    out_hbm.at[c, pl.ds(base, ROWS_PER_TILE)])

    @pl.when(s == NS - 1)
    def _():
        pltpu.sync_copy(agg_sh.at[pl.ds(NS * ROWS_PER_TILE, ROWS_TAIL)],
                        out_hbm.at[c, pl.ds(NS * ROWS_PER_TILE, ROWS_TAIL)])


@jax.jit
def _sc_segsum(x, ei, zeros):
    mesh = plsc.VectorSubcoreMesh(core_axis_name="c", subcore_axis_name="s")
    return pl.kernel(
        _segsum_body,
        out_type=jax.ShapeDtypeStruct((NC, N_NODES, DIM), jnp.float32),
        mesh=mesh,
        scratch_types=[
            pltpu.VMEM((2, PHB, EB), jnp.int32),
            pltpu.VMEM((2, PHB, EB), jnp.int32),
            pltpu.VMEM((NBUF, EB, DIM), jnp.float32),
            pltpu.VMEM_SHARED((AGG_ROWS, DIM), jnp.float32),
            [pltpu.SemaphoreType.DMA] * NBUF,
            [pltpu.SemaphoreType.DMA] * 2,
            [pltpu.SemaphoreType.DMA] * 2,
        ],
    )(x, ei, zeros)


ROW_BLK = 2000


def _mlp_body(agg_ref, wa_ref, wb_ref, wc_ref, l_ref,
              hout_ref, head_ref, acc_ref):
    i = pl.program_id(0)
    z = agg_ref[0] + agg_ref[1]
    z = jnp.maximum(jnp.dot(z, wa_ref[...], preferred_element_type=jnp.float32), 0.0)
    z = jnp.maximum(jnp.dot(z, wb_ref[...], preferred_element_type=jnp.float32), 0.0)
    z = jnp.maximum(jnp.dot(z, wc_ref[...], preferred_element_type=jnp.float32), 0.0)
    hout_ref[...] = z

    @pl.when(i == 0)
    def _():
        acc_ref[...] = jnp.zeros_like(acc_ref)

    acc_ref[...] += jnp.sum(z, axis=0, keepdims=True)

    @pl.when(i == pl.num_programs(0) - 1)
    def _():
        head_ref[...] = jnp.dot(acc_ref[...] / N_NODES, l_ref[...],
                                preferred_element_type=jnp.float32)


def _final_body(agg_ref, wa_ref, wb_ref, wc_ref, l_ref,
                h0_ref, h1_ref, h2_ref, h3_ref, out_ref, acc_ref):
    i = pl.program_id(0)
    z = agg_ref[0] + agg_ref[1]
    z = jnp.maximum(jnp.dot(z, wa_ref[...], preferred_element_type=jnp.float32), 0.0)
    z = jnp.maximum(jnp.dot(z, wb_ref[...], preferred_element_type=jnp.float32), 0.0)
    z = jnp.maximum(jnp.dot(z, wc_ref[...], preferred_element_type=jnp.float32), 0.0)

    @pl.when(i == 0)
    def _():
        acc_ref[...] = jnp.zeros_like(acc_ref)

    acc_ref[...] += jnp.sum(z, axis=0, keepdims=True)

    @pl.when(i == pl.num_programs(0) - 1)
    def _():
        head = jnp.dot(acc_ref[...] / N_NODES, l_ref[...],
                       preferred_element_type=jnp.float32)
        total = head + h0_ref[...] + h1_ref[...] + h2_ref[...] + h3_ref[...]
        out_ref[...] = jax.nn.sigmoid(total)


def _tc_mlp(agg, wa, wb, wc, l):
    grid = N_NODES // ROW_BLK
    return pl.pallas_call(
        _mlp_body,
        grid=(grid,),
        in_specs=[
            pl.BlockSpec((NC, ROW_BLK, DIM), lambda i: (0, i, 0)),
            pl.BlockSpec((DIM, DIM), lambda i: (0, 0)),
            pl.BlockSpec((DIM, DIM), lambda i: (0, 0)),
            pl.BlockSpec((DIM, DIM), lambda i: (0, 0)),
            pl.BlockSpec((DIM, 1), lambda i: (0, 0)),
        ],
        out_specs=[
            pl.BlockSpec((ROW_BLK, DIM), lambda i: (i, 0)),
            pl.BlockSpec((1, 1), lambda i: (0, 0)),
        ],
        out_shape=[
            jax.ShapeDtypeStruct((N_NODES, DIM), jnp.float32),
            jax.ShapeDtypeStruct((1, 1), jnp.float32),
        ],
        scratch_shapes=[pltpu.VMEM((1, DIM), jnp.float32)],
    )(agg, wa, wb, wc, l)


def _tc_final(agg, wa, wb, wc, l, h0, h1, h2, h3):
    grid = N_NODES // ROW_BLK
    hspec = pl.BlockSpec((1, 1), lambda i: (0, 0))
    return pl.pallas_call(
        _final_body,
        grid=(grid,),
        in_specs=[
            pl.BlockSpec((NC, ROW_BLK, DIM), lambda i: (0, i, 0)),
            pl.BlockSpec((DIM, DIM), lambda i: (0, 0)),
            pl.BlockSpec((DIM, DIM), lambda i: (0, 0)),
            pl.BlockSpec((DIM, DIM), lambda i: (0, 0)),
            pl.BlockSpec((DIM, 1), lambda i: (0, 0)),
            hspec, hspec, hspec, hspec,
        ],
        out_specs=pl.BlockSpec((1, 1), lambda i: (0, 0)),
        out_shape=jax.ShapeDtypeStruct((1, 1), jnp.float32),
        scratch_shapes=[pltpu.VMEM((1, DIM), jnp.float32)],
    )(agg, wa, wb, wc, l, h0, h1, h2, h3)


def kernel(x, edge_index, Wa, Wb, Wc, L):
    ei = edge_index.reshape(2, NBLK, EB)
    zeros = jnp.zeros((AGG_ROWS, DIM), jnp.float32)

    h = x
    heads = []
    for i in range(4):
        agg = _sc_segsum(h, ei, zeros)
        h, head = _tc_mlp(agg, Wa[i], Wb[i], Wc[i], L[i])
        heads.append(head)

    agg = _sc_segsum(h, ei, zeros)
    out = _tc_final(agg, Wa[4], Wb[4], Wc[4], L[4], *heads)
    return out.reshape((1,))
